# baseline (device time: 55676 ns/iter reference)
import jax
import jax.numpy as jnp
from jax import lax
from jax.experimental import pallas as pl
from jax.experimental.pallas import tpu as pltpu

N_DEV = 16
N_SUB = 8
N_HOPS = N_SUB - 1
B = 2
S = 128
BLK = 64
HQ = 4
DH = 64
D_MODEL = 512
D_QK = HQ * DH


def _body(x_ref, wq_ref, k_ref, v_ref, wo_ref, out_ref,
          comm_ref, send_sems, recv_sems):
    me = lax.axis_index("i")
    right = lax.rem(me + 2, N_DEV)

    comm_ref[0, 0] = k_ref[...]
    comm_ref[0, 1] = v_ref[...]

    for h in range(N_HOPS):
        rdma = pltpu.make_async_remote_copy(
            src_ref=comm_ref.at[h],
            dst_ref=comm_ref.at[h + 1],
            send_sem=send_sems.at[h],
            recv_sem=recv_sems.at[h],
            device_id=(right,),
            device_id_type=pl.DeviceIdType.MESH,
        )
        rdma.start()
        rdma.wait()

    wq = wq_ref[...].astype(jnp.bfloat16)
    wo = wo_ref[...].astype(jnp.bfloat16)
    for b in range(B):
        xb = x_ref[b].astype(jnp.bfloat16)
        q_all = lax.dot(xb, wq, preferred_element_type=jnp.float32)
        q_all = (q_all * 0.125).astype(jnp.bfloat16)
        row_blocks = []
        for i in range(2):
            head_ctx = []
            for hh in range(HQ):
                q = q_all[i * BLK:(i + 1) * BLK, hh * DH:(hh + 1) * DH]
                kc = jnp.concatenate(
                    [comm_ref[s, 0, b, hh, :, i * BLK:(i + 1) * BLK]
                     for s in range(N_SUB)], axis=1)
                vc = jnp.concatenate(
                    [comm_ref[s, 1, b, hh, :, i * BLK:(i + 1) * BLK]
                     for s in range(N_SUB)], axis=1)
                scores = lax.dot(q, kc, preferred_element_type=jnp.float32)
                m = jnp.max(scores, axis=1, keepdims=True)
                e = jnp.exp(scores - m)
                den = jnp.sum(e, axis=1, keepdims=True)
                w = (e / den).astype(jnp.bfloat16)
                ctx = lax.dot_general(
                    w, vc, (((1,), (1,)), ((), ())),
                    preferred_element_type=jnp.float32)
                head_ctx.append(ctx.astype(jnp.bfloat16))
            row_blocks.append(jnp.concatenate(head_ctx, axis=1))
        ctx_b = jnp.concatenate(row_blocks, axis=0)
        out_ref[b] = lax.dot(ctx_b, wo, preferred_element_type=jnp.float32)


def kernel(x, Wq, K_ext, V_ext, Wo):
    k_t = jnp.transpose(K_ext, (0, 2, 3, 1)).astype(jnp.bfloat16)
    v_t = jnp.transpose(V_ext, (0, 2, 3, 1)).astype(jnp.bfloat16)

    return pl.pallas_call(
        _body,
        out_shape=jax.ShapeDtypeStruct((B, S, D_MODEL), jnp.float32),
        in_specs=[pl.BlockSpec(memory_space=pltpu.VMEM)] * 5,
        out_specs=pl.BlockSpec(memory_space=pltpu.VMEM),
        scratch_shapes=[
            pltpu.VMEM((N_SUB, 2, B, HQ, DH, S), jnp.bfloat16),
            pltpu.SemaphoreType.DMA((N_HOPS,)),
            pltpu.SemaphoreType.DMA((N_HOPS,)),
        ],
    )(x, Wq, k_t, v_t, Wo)


# device time: 36791 ns/iter; 1.5133x vs baseline; 1.5133x over previous
import jax
import jax.numpy as jnp
from jax import lax
from jax.experimental import pallas as pl
from jax.experimental.pallas import tpu as pltpu

N_DEV = 16
N_SUB = 8
B = 2
S = 128
BLK = 64
HQ = 4
DH = 64
D_MODEL = 512


def _body(x_ref, wq_ref, k_ref, v_ref, wo_ref, out_ref,
          comm_ref, send_sems, recv_sems):
    me = lax.axis_index("i")
    c = lax.rem(me, 2)
    z = lax.div(me, 4)
    diag = lax.div(lax.rem(me, 4), 2)
    v_me = 2 * z + diag

    comm_ref[0, 0] = k_ref[...]
    comm_ref[0, 1] = v_ref[...]

    rdmas = []
    for k in range(1, N_SUB):
        vt = lax.rem(v_me + k, N_SUB)
        target = 4 * lax.div(vt, 2) + 2 * lax.rem(vt, 2) + c
        slot = N_SUB - k
        for kv in range(2):
            rdma = pltpu.make_async_remote_copy(
                src_ref=(k_ref if kv == 0 else v_ref),
                dst_ref=comm_ref.at[slot, kv],
                send_sem=send_sems.at[kv, k - 1],
                recv_sem=recv_sems.at[kv, slot - 1],
                device_id=(target,),
                device_id_type=pl.DeviceIdType.MESH,
            )
            rdma.start()
            rdmas.append(rdma)

    wq = wq_ref[...].astype(jnp.bfloat16)
    wo = wo_ref[...].astype(jnp.bfloat16)
    q_bf = []
    for b in range(B):
        xb = x_ref[b].astype(jnp.bfloat16)
        q_all = lax.dot(xb, wq, preferred_element_type=jnp.float32)
        q_bf.append((q_all * 0.125).astype(jnp.bfloat16))

    for rdma in rdmas:
        rdma.wait_recv()

    for b in range(B):
        row_blocks = []
        for i in range(2):
            head_ctx = []
            for hh in range(HQ):
                q = q_bf[b][i * BLK:(i + 1) * BLK, hh * DH:(hh + 1) * DH]
                kc = jnp.concatenate(
                    [comm_ref[s, 0, b, hh, :, i * BLK:(i + 1) * BLK]
                     for s in range(N_SUB)], axis=1)
                vc = jnp.concatenate(
                    [comm_ref[s, 1, b, hh, :, i * BLK:(i + 1) * BLK]
                     for s in range(N_SUB)], axis=1)
                scores = lax.dot(q, kc, preferred_element_type=jnp.float32)
                m = jnp.max(scores, axis=1, keepdims=True)
                e = jnp.exp(scores - m)
                den = jnp.sum(e, axis=1, keepdims=True)
                w = (e / den).astype(jnp.bfloat16)
                ctx = lax.dot_general(
                    w, vc, (((1,), (1,)), ((), ())),
                    preferred_element_type=jnp.float32)
                head_ctx.append(ctx.astype(jnp.bfloat16))
            row_blocks.append(jnp.concatenate(head_ctx, axis=1))
        ctx_b = jnp.concatenate(row_blocks, axis=0)
        out_ref[b] = lax.dot(ctx_b, wo, preferred_element_type=jnp.float32)

    for rdma in rdmas:
        rdma.wait_send()


def kernel(x, Wq, K_ext, V_ext, Wo):
    k_t = jnp.transpose(K_ext, (0, 2, 3, 1)).astype(jnp.bfloat16)
    v_t = jnp.transpose(V_ext, (0, 2, 3, 1)).astype(jnp.bfloat16)

    return pl.pallas_call(
        _body,
        out_shape=jax.ShapeDtypeStruct((B, S, D_MODEL), jnp.float32),
        in_specs=[pl.BlockSpec(memory_space=pltpu.VMEM)] * 5,
        out_specs=pl.BlockSpec(memory_space=pltpu.VMEM),
        scratch_shapes=[
            pltpu.VMEM((N_SUB, 2, B, HQ, DH, S), jnp.bfloat16),
            pltpu.SemaphoreType.DMA((2, N_SUB - 1)),
            pltpu.SemaphoreType.DMA((2, N_SUB - 1)),
        ],
    )(x, Wq, k_t, v_t, Wo)


# device time: 28228 ns/iter; 1.9724x vs baseline; 1.3034x over previous
import jax
import jax.numpy as jnp
from jax import lax
from jax.experimental import pallas as pl
from jax.experimental.pallas import tpu as pltpu

N_DEV = 16
N_SUB = 8
B = 2
S = 128
BLK = 64
HQ = 4
DH = 64
D_MODEL = 512

_HALF1 = (0, 5, 6, 7)
_HALF2 = (1, 2, 3, 4)


def _attend(q_bf, comm_ref, slots):
    out = {}
    for b in range(B):
        for i in range(2):
            nums, dens = [], []
            for hh in range(HQ):
                q = q_bf[b][i * BLK:(i + 1) * BLK, hh * DH:(hh + 1) * DH]
                kc = jnp.concatenate(
                    [comm_ref[s, 0, b, hh, :, i * BLK:(i + 1) * BLK]
                     for s in slots], axis=1)
                vc = jnp.concatenate(
                    [comm_ref[s, 1, b, hh, :, i * BLK:(i + 1) * BLK]
                     for s in slots], axis=1)
                scores = lax.dot(q, kc, preferred_element_type=jnp.float32)
                e = jnp.exp(scores)
                den = jnp.sum(e, axis=1, keepdims=True)
                num = lax.dot_general(
                    e.astype(jnp.bfloat16), vc, (((1,), (1,)), ((), ())),
                    preferred_element_type=jnp.float32)
                nums.append(num)
                dens.append(den)
            out[(b, i)] = (nums, dens)
    return out


def _body(x_ref, wq_ref, k_ref, v_ref, wo_ref, out_ref,
          comm_ref, send_sems, recv_sems):
    me = lax.axis_index("i")
    c = lax.rem(me, 2)
    z = lax.div(me, 4)
    diag = lax.div(lax.rem(me, 4), 2)
    v_me = 2 * z + diag

    def peer(k):
        vt = lax.rem(v_me + k, N_SUB)
        return 4 * lax.div(vt, 2) + 2 * lax.rem(vt, 2) + c

    barrier_sem = pltpu.get_barrier_semaphore()
    for k in range(1, N_SUB):
        pl.semaphore_signal(barrier_sem, inc=1, device_id=(peer(k),),
                            device_id_type=pl.DeviceIdType.MESH)
    pl.semaphore_wait(barrier_sem, N_SUB - 1)

    comm_ref[0, 0] = k_ref[...]
    comm_ref[0, 1] = v_ref[...]

    rdmas = {}
    for k in range(1, N_SUB):
        slot = N_SUB - k
        pair = []
        for kv in range(2):
            rdma = pltpu.make_async_remote_copy(
                src_ref=(k_ref if kv == 0 else v_ref),
                dst_ref=comm_ref.at[slot, kv],
                send_sem=send_sems.at[kv, k - 1],
                recv_sem=recv_sems.at[kv, slot - 1],
                device_id=(peer(k),),
                device_id_type=pl.DeviceIdType.MESH,
            )
            rdma.start()
            pair.append(rdma)
        rdmas[slot] = pair

    wq = wq_ref[...].astype(jnp.bfloat16)
    wo = wo_ref[...].astype(jnp.bfloat16)
    q_bf = []
    for b in range(B):
        xb = x_ref[b].astype(jnp.bfloat16)
        q_all = lax.dot(xb, wq, preferred_element_type=jnp.float32)
        q_bf.append((q_all * 0.125).astype(jnp.bfloat16))

    for s in _HALF1[1:]:
        for r in rdmas[s]:
            r.wait_recv()
    acc1 = _attend(q_bf, comm_ref, _HALF1)

    for s in _HALF2:
        for r in rdmas[s]:
            r.wait_recv()
    acc2 = _attend(q_bf, comm_ref, _HALF2)

    for b in range(B):
        row_blocks = []
        for i in range(2):
            nums1, dens1 = acc1[(b, i)]
            nums2, dens2 = acc2[(b, i)]
            head_ctx = [
                ((nums1[hh] + nums2[hh]) / (dens1[hh] + dens2[hh]))
                .astype(jnp.bfloat16)
                for hh in range(HQ)
            ]
            row_blocks.append(jnp.concatenate(head_ctx, axis=1))
        ctx_b = jnp.concatenate(row_blocks, axis=0)
        out_ref[b] = lax.dot(ctx_b, wo, preferred_element_type=jnp.float32)

    for pair in rdmas.values():
        for r in pair:
            r.wait_send()


def kernel(x, Wq, K_ext, V_ext, Wo):
    k_t = jnp.transpose(K_ext, (0, 2, 3, 1)).astype(jnp.bfloat16)
    v_t = jnp.transpose(V_ext, (0, 2, 3, 1)).astype(jnp.bfloat16)

    return pl.pallas_call(
        _body,
        out_shape=jax.ShapeDtypeStruct((B, S, D_MODEL), jnp.float32),
        in_specs=[pl.BlockSpec(memory_space=pltpu.VMEM)] * 5,
        out_specs=pl.BlockSpec(memory_space=pltpu.VMEM),
        scratch_shapes=[
            pltpu.VMEM((N_SUB, 2, B, HQ, DH, S), jnp.bfloat16),
            pltpu.SemaphoreType.DMA((2, N_SUB - 1)),
            pltpu.SemaphoreType.DMA((2, N_SUB - 1)),
        ],
        compiler_params=pltpu.CompilerParams(collective_id=0),
    )(x, Wq, k_t, v_t, Wo)
